# Initial kernel scaffold; baseline (speedup 1.0000x reference)
import jax
import jax.numpy as jnp
from jax.experimental import pallas as pl


def kernel(x, historical_data, W1, b1, W2, b2, gamma, beta):
    raise NotImplementedError("write your pallas kernel here")


# speedup vs baseline: 1.7679x; 1.7679x over previous
import functools

import jax
import jax.numpy as jnp
from jax import lax
from jax.experimental import pallas as pl
from jax.experimental.pallas import tpu as pltpu
from jax.experimental.pallas import tpu_sc as plsc

B, D, KH, KN = 1024, 16, 100000, 5
BT, KT = 256, 2048
KHP = 102400
NEG = float(-3.0e38)
IMAX = 2**31 - 1
NC, NS = 2, 16
NW = NC * NS
ROWS_PER_W = (B * KN) // NW


def _scan_kernel(xl_ref, ht_ref, v_ref, i_ref):
    k = pl.program_id(1)
    xl = xl_ref[...]
    ht = ht_ref[...]
    xn = jnp.maximum(jnp.sqrt(jnp.sum(xl * xl, axis=1, keepdims=True)), 1e-8)
    hn = jnp.maximum(jnp.sqrt(jnp.sum(ht * ht, axis=0, keepdims=True)), 1e-8)
    dot = lax.dot_general(xl, ht, (((1,), (0,)), ((), ())),
                          preferred_element_type=jnp.float32)
    cols = lax.broadcasted_iota(jnp.int32, (BT, KT), 1) + k * KT
    v = dot / (xn * hn)
    v = jnp.where(cols < KH, v, NEG)

    tv, ti = [], []
    for _ in range(KN):
        m = jnp.max(v, axis=1, keepdims=True)
        sel = jnp.min(jnp.where(v == m, cols, IMAX), axis=1, keepdims=True)
        tv.append(m)
        ti.append(sel)
        v = jnp.where(cols == sel, NEG, v)
    tv = jnp.concatenate(tv, axis=1)
    ti = jnp.concatenate(ti, axis=1)

    @pl.when(k == 0)
    def _():
        v_ref[...] = tv
        i_ref[...] = ti

    @pl.when(k > 0)
    def _():
        cv = jnp.concatenate([v_ref[...], tv], axis=1)
        ci = jnp.concatenate([i_ref[...], ti], axis=1)
        nv, ni = [], []
        for _ in range(KN):
            m = jnp.max(cv, axis=1, keepdims=True)
            elig = cv == m
            sel = jnp.min(jnp.where(elig, ci, IMAX), axis=1, keepdims=True)
            nv.append(m)
            ni.append(sel)
            cv = jnp.where(elig & (ci == sel), NEG, cv)
        v_ref[...] = jnp.concatenate(nv, axis=1)
        i_ref[...] = jnp.concatenate(ni, axis=1)


_topk = pl.pallas_call(
    _scan_kernel,
    grid=(B // BT, KHP // KT),
    in_specs=[
        pl.BlockSpec((BT, D), lambda b, k: (b, 0)),
        pl.BlockSpec((D, KT), lambda b, k: (0, k)),
    ],
    out_specs=[
        pl.BlockSpec((BT, KN), lambda b, k: (b, 0)),
        pl.BlockSpec((BT, KN), lambda b, k: (b, 0)),
    ],
    out_shape=[
        jax.ShapeDtypeStruct((B, KN), jnp.float32),
        jax.ShapeDtypeStruct((B, KN), jnp.int32),
    ],
)


GROUP = 128 // D
CH = ROWS_PER_W // 2


@functools.lru_cache(maxsize=1)
def _sc_gather():
    mesh = plsc.VectorSubcoreMesh(core_axis_name="c", subcore_axis_name="s")

    @functools.partial(
        pl.kernel,
        mesh=mesh,
        out_type=jax.ShapeDtypeStruct((B * KN, 128), jnp.float32),
        scratch_types=[
            pltpu.VMEM((2, CH), jnp.int32),
            pltpu.VMEM((ROWS_PER_W, 128), jnp.float32),
            pltpu.SemaphoreType.DMA,
        ],
    )
    def gather(table_hbm, idx_hbm, out_hbm, idx_v, rows_v, sem):
        wid = lax.axis_index("s") * NC + lax.axis_index("c")
        base = wid * ROWS_PER_W
        pltpu.sync_copy(idx_hbm.at[wid], idx_v)
        cp0 = pltpu.async_copy(table_hbm.at[idx_v.at[0]],
                               rows_v.at[pl.ds(0, CH)], sem)
        cp1 = pltpu.async_copy(table_hbm.at[idx_v.at[1]],
                               rows_v.at[pl.ds(CH, CH)], sem)
        cp0.wait()
        cp1.wait()
        pltpu.sync_copy(rows_v, out_hbm.at[pl.ds(base, ROWS_PER_W)])

    return gather


def _fuse_kernel(tv_ref, ti_ref, rows_ref, xl_ref, w1a_ref, w1b_ref, b1_ref,
                 w2_ref, b2_ref, g_ref, bt_ref, o_ref):
    tv = tv_ref[...]
    m = jnp.max(tv, axis=1, keepdims=True)
    e = jnp.exp(tv - m)
    w = e / jnp.sum(e, axis=1, keepdims=True)
    rows = rows_ref[...]
    ti = ti_ref[...]
    ws = jnp.zeros((B, D), dtype=jnp.float32)
    for j in range(KN):
        sub = lax.rem(ti[:, j:j + 1], GROUP)
        hj = jnp.zeros((B, D), dtype=jnp.float32)
        for s in range(GROUP):
            sl = rows[:, j * 128 + s * D:j * 128 + (s + 1) * D]
            hj = jnp.where(sub == s, sl, hj)
        ws = ws + w[:, j:j + 1] * hj
    xl = xl_ref[...]
    dn = (((1,), (0,)), ((), ()))
    h = lax.dot_general(xl, w1a_ref[...], dn,
                        preferred_element_type=jnp.float32,
                        precision=lax.Precision.HIGHEST)
    h = h + lax.dot_general(ws, w1b_ref[...], dn,
                            preferred_element_type=jnp.float32,
                            precision=lax.Precision.HIGHEST)
    h = jnp.maximum(h + b1_ref[...], 0.0)
    h2 = lax.dot_general(h, w2_ref[...], dn,
                         preferred_element_type=jnp.float32,
                         precision=lax.Precision.HIGHEST) + b2_ref[...]
    mu = jnp.mean(h2, axis=1, keepdims=True)
    var = jnp.mean((h2 - mu) ** 2, axis=1, keepdims=True)
    o_ref[...] = (h2 - mu) / jnp.sqrt(var + 1e-5) * g_ref[...] + bt_ref[...]


_fuse = pl.pallas_call(
    _fuse_kernel,
    out_shape=jax.ShapeDtypeStruct((B, D), jnp.float32),
)


def kernel(x, historical_data, W1, b1, W2, b2, gamma, beta):
    xl = x[:, -1, :]
    ht = jnp.pad(historical_data.T, ((0, 0), (0, KHP - KH)))
    tv, ti = _topk(xl, ht)
    table8 = historical_data.reshape(KH // GROUP, 128)
    idx_g = (ti // GROUP).reshape(NW, 2, CH)
    rows = _sc_gather()(table8, idx_g)
    w1a = W1[:D]
    w1b = W1[D:].reshape(KN, D, 2 * D).sum(axis=0)
    out = _fuse(tv, ti, rows.reshape(B, KN * 128), xl, w1a, w1b,
                b1.reshape(1, -1), W2, b2.reshape(1, -1),
                gamma.reshape(1, -1), beta.reshape(1, -1))
    return out


# speedup vs baseline: 4.5885x; 2.5955x over previous
import functools

import jax
import jax.numpy as jnp
from jax import lax
from jax.experimental import pallas as pl
from jax.experimental.pallas import tpu as pltpu
from jax.experimental.pallas import tpu_sc as plsc

B, D, KH, KN = 1024, 16, 100000, 5
BT, KT = 256, 2048
KHP = 102400
NEG = float(-3.0e38)
IMAX = 2**31 - 1
NC, NS = 2, 16
NW = NC * NS
ROWS_PER_W = (B * KN) // NW


G = 128
NG = KHP // G
GPT = KT // G
NT = KHP // KT


def _scan_kernel(xl_ref, ht_ref, gm_ref):
    k = pl.program_id(0)
    xl = xl_ref[...]
    ht = ht_ref[...]
    hn2 = jnp.sum(ht * ht, axis=0, keepdims=True)
    inv_hn = 1.0 / jnp.maximum(jnp.sqrt(hn2), 1e-8)
    col = lax.broadcasted_iota(jnp.int32, (1, KT), 1) + k * KT
    bias = jnp.where(col < KH, 0.0, NEG)
    dot = lax.dot_general(xl, ht, (((1,), (0,)), ((), ())),
                          preferred_element_type=jnp.float32)
    v = dot * inv_hn + bias
    gm_ref[...] = jnp.max(v.reshape(B, GPT, G), axis=2)[None]


_scan = pl.pallas_call(
    _scan_kernel,
    grid=(NT,),
    in_specs=[
        pl.BlockSpec((B, D), lambda k: (0, 0)),
        pl.BlockSpec((D, KT), lambda k: (0, k)),
    ],
    out_specs=pl.BlockSpec((1, B, GPT), lambda k: (k, 0, 0)),
    out_shape=jax.ShapeDtypeStruct((NT, B, GPT), jnp.float32),
)


def _gtopk_kernel(gm_ref, gi_ref):
    v = gm_ref[...]
    cols = lax.broadcasted_iota(jnp.int32, (B, NG), 1)
    gis = []
    for _ in range(KN):
        m = jnp.max(v, axis=1, keepdims=True)
        sel = jnp.min(jnp.where(v == m, cols, IMAX), axis=1, keepdims=True)
        gis.append(sel)
        v = jnp.where(cols == sel, NEG, v)
    gi_ref[...] = jnp.concatenate(gis, axis=1)


_gtopk = pl.pallas_call(
    _gtopk_kernel,
    out_shape=jax.ShapeDtypeStruct((B, KN), jnp.int32),
)

GW = (B * KN) // NW
NCH = 4
CHG = GW // NCH


@functools.lru_cache(maxsize=1)
def _sc_group_gather():
    mesh = plsc.VectorSubcoreMesh(core_axis_name="c", subcore_axis_name="s")

    @functools.partial(
        pl.kernel,
        mesh=mesh,
        out_type=jax.ShapeDtypeStruct((B * KN, D * G), jnp.float32),
        scratch_types=[
            pltpu.VMEM((CHG,), jnp.int32),
            pltpu.VMEM((CHG, D * G), jnp.float32),
            pltpu.SemaphoreType.DMA,
        ],
    )
    def gather(table_hbm, idx_hbm, out_hbm, idx_v, buf_v, sem):
        wid = lax.axis_index("s") * NC + lax.axis_index("c")
        for c in range(NCH):
            pltpu.sync_copy(idx_hbm.at[wid, c], idx_v)
            pltpu.async_copy(table_hbm.at[idx_v], buf_v, sem).wait()
            pltpu.sync_copy(buf_v, out_hbm.at[pl.ds(wid * GW + c * CHG, CHG)])

    return gather


SB = 640


def _sims_kernel(cand_ref, xl5_ref, dot_ref, hn2_ref):
    c = cand_ref[...].reshape(SB, D, G)
    x3 = xl5_ref[...].reshape(SB, D, 1)
    dot_ref[...] = jnp.sum(c * x3, axis=1)
    hn2_ref[...] = jnp.sum(c * c, axis=1)


_sims = pl.pallas_call(
    _sims_kernel,
    grid=((B * KN) // SB,),
    in_specs=[
        pl.BlockSpec((SB, D * G), lambda i: (i, 0)),
        pl.BlockSpec((SB, D), lambda i: (i, 0)),
    ],
    out_specs=[
        pl.BlockSpec((SB, G), lambda i: (i, 0)),
        pl.BlockSpec((SB, G), lambda i: (i, 0)),
    ],
    out_shape=[
        jax.ShapeDtypeStruct((B * KN, G), jnp.float32),
        jax.ShapeDtypeStruct((B * KN, G), jnp.float32),
    ],
)


def _ftopk_kernel(dot_ref, hn2_ref, gi_ref, xl_ref, tv_ref, ti_ref):
    xl = xl_ref[...]
    xn = jnp.maximum(jnp.sqrt(jnp.sum(xl * xl, axis=1, keepdims=True)), 1e-8)
    hn = jnp.maximum(jnp.sqrt(hn2_ref[...]), 1e-8)
    v = dot_ref[...] / (xn * hn)
    gi = gi_ref[...]
    pos = lax.broadcasted_iota(jnp.int32, (B, KN * G), 1)
    pj = pos // G
    gsel = jnp.zeros((B, KN * G), dtype=jnp.int32)
    for j in range(KN):
        gsel = jnp.where(pj == j, gi[:, j:j + 1], gsel)
    gidx = gsel * G + (pos - pj * G)
    v = jnp.where(gidx < KH, v, NEG)
    tvs, tis = [], []
    for _ in range(KN):
        m = jnp.max(v, axis=1, keepdims=True)
        sel = jnp.min(jnp.where(v == m, gidx, IMAX), axis=1, keepdims=True)
        tvs.append(m)
        tis.append(sel)
        v = jnp.where(gidx == sel, NEG, v)
    tv_ref[...] = jnp.concatenate(tvs, axis=1)
    ti_ref[...] = jnp.concatenate(tis, axis=1)


_ftopk = pl.pallas_call(
    _ftopk_kernel,
    out_shape=[
        jax.ShapeDtypeStruct((B, KN), jnp.float32),
        jax.ShapeDtypeStruct((B, KN), jnp.int32),
    ],
)


GROUP = 128 // D
CH = ROWS_PER_W // 2


@functools.lru_cache(maxsize=1)
def _sc_gather():
    mesh = plsc.VectorSubcoreMesh(core_axis_name="c", subcore_axis_name="s")

    @functools.partial(
        pl.kernel,
        mesh=mesh,
        out_type=jax.ShapeDtypeStruct((B * KN, 128), jnp.float32),
        scratch_types=[
            pltpu.VMEM((2, CH), jnp.int32),
            pltpu.VMEM((ROWS_PER_W, 128), jnp.float32),
            pltpu.SemaphoreType.DMA,
        ],
    )
    def gather(table_hbm, idx_hbm, out_hbm, idx_v, rows_v, sem):
        wid = lax.axis_index("s") * NC + lax.axis_index("c")
        base = wid * ROWS_PER_W
        pltpu.sync_copy(idx_hbm.at[wid], idx_v)
        cp0 = pltpu.async_copy(table_hbm.at[idx_v.at[0]],
                               rows_v.at[pl.ds(0, CH)], sem)
        cp1 = pltpu.async_copy(table_hbm.at[idx_v.at[1]],
                               rows_v.at[pl.ds(CH, CH)], sem)
        cp0.wait()
        cp1.wait()
        pltpu.sync_copy(rows_v, out_hbm.at[pl.ds(base, ROWS_PER_W)])

    return gather


def _fuse_kernel(tv_ref, ti_ref, rows_ref, xl_ref, w1a_ref, w1b_ref, b1_ref,
                 w2_ref, b2_ref, g_ref, bt_ref, o_ref):
    tv = tv_ref[...]
    m = jnp.max(tv, axis=1, keepdims=True)
    e = jnp.exp(tv - m)
    w = e / jnp.sum(e, axis=1, keepdims=True)
    rows = rows_ref[...]
    ti = ti_ref[...]
    ws = jnp.zeros((B, D), dtype=jnp.float32)
    for j in range(KN):
        sub = lax.rem(ti[:, j:j + 1], GROUP)
        hj = jnp.zeros((B, D), dtype=jnp.float32)
        for s in range(GROUP):
            sl = rows[:, j * 128 + s * D:j * 128 + (s + 1) * D]
            hj = jnp.where(sub == s, sl, hj)
        ws = ws + w[:, j:j + 1] * hj
    xl = xl_ref[...]
    dn = (((1,), (0,)), ((), ()))
    h = lax.dot_general(xl, w1a_ref[...], dn,
                        preferred_element_type=jnp.float32,
                        precision=lax.Precision.HIGHEST)
    h = h + lax.dot_general(ws, w1b_ref[...], dn,
                            preferred_element_type=jnp.float32,
                            precision=lax.Precision.HIGHEST)
    h = jnp.maximum(h + b1_ref[...], 0.0)
    h2 = lax.dot_general(h, w2_ref[...], dn,
                         preferred_element_type=jnp.float32,
                         precision=lax.Precision.HIGHEST) + b2_ref[...]
    mu = jnp.mean(h2, axis=1, keepdims=True)
    var = jnp.mean((h2 - mu) ** 2, axis=1, keepdims=True)
    o_ref[...] = (h2 - mu) / jnp.sqrt(var + 1e-5) * g_ref[...] + bt_ref[...]


_fuse = pl.pallas_call(
    _fuse_kernel,
    out_shape=jax.ShapeDtypeStruct((B, D), jnp.float32),
)


def kernel(x, historical_data, W1, b1, W2, b2, gamma, beta):
    xl = x[:, -1, :]
    ht = jnp.pad(historical_data.T, ((0, 0), (0, KHP - KH)))
    gm = _scan(xl, ht)
    gi = _gtopk(gm.transpose(1, 0, 2).reshape(B, NG))
    table_g = ht.reshape(D, NG, G).transpose(1, 0, 2).reshape(NG, D * G)
    cand = _sc_group_gather()(table_g, gi.reshape(NW, NCH, CHG))
    xl5 = jnp.repeat(xl, KN, axis=0)
    dotc, hn2c = _sims(cand, xl5)
    tv, ti = _ftopk(dotc.reshape(B, KN * G), hn2c.reshape(B, KN * G), gi, xl)
    table8 = historical_data.reshape(KH // GROUP, 128)
    idx_g = (ti // GROUP).reshape(NW, 2, CH)
    rows = _sc_gather()(table8, idx_g)
    w1a = W1[:D]
    w1b = W1[D:].reshape(KN, D, 2 * D).sum(axis=0)
    out = _fuse(tv, ti, rows.reshape(B, KN * 128), xl, w1a, w1b,
                b1.reshape(1, -1), W2, b2.reshape(1, -1),
                gamma.reshape(1, -1), beta.reshape(1, -1))
    return out


# speedup vs baseline: 4.6546x; 1.0144x over previous
import functools

import jax
import jax.numpy as jnp
from jax import lax
from jax.experimental import pallas as pl
from jax.experimental.pallas import tpu as pltpu
from jax.experimental.pallas import tpu_sc as plsc

B, D, KH, KN = 1024, 16, 100000, 5
BT, KT = 256, 2048
KHP = 102400
NEG = float(-3.0e38)
IMAX = 2**31 - 1
NC, NS = 2, 16
NW = NC * NS
ROWS_PER_W = (B * KN) // NW


G = 128
NG = KHP // G
GPT = KT // G
NT = KHP // KT


def _norm_kernel(ht_ref, o_ref):
    ht = ht_ref[...]
    hn2 = jnp.sum(ht * ht, axis=0, keepdims=True)
    inv = 1.0 / jnp.maximum(jnp.sqrt(hn2), 1e-8)
    o_ref[...] = ht * inv


_norm = pl.pallas_call(
    _norm_kernel,
    out_shape=jax.ShapeDtypeStruct((D, KHP), jnp.float32),
)


def _scan_kernel(xl_ref, ht_ref, gm_ref):
    xl = xl_ref[...]
    dot = lax.dot_general(xl, ht_ref[...], (((1,), (0,)), ((), ())),
                          preferred_element_type=jnp.float32)
    gm_ref[...] = jnp.max(dot.reshape(B, GPT, G), axis=2)[None]


_scan = pl.pallas_call(
    _scan_kernel,
    grid=(NT,),
    in_specs=[
        pl.BlockSpec((B, D), lambda k: (0, 0)),
        pl.BlockSpec((D, KT), lambda k: (0, k)),
    ],
    out_specs=pl.BlockSpec((1, B, GPT), lambda k: (k, 0, 0)),
    out_shape=jax.ShapeDtypeStruct((NT, B, GPT), jnp.float32),
)


def _gtopk_kernel(gm_ref, gi_ref):
    v = gm_ref[...]
    cols = lax.broadcasted_iota(jnp.int32, (B, NG), 1)
    v = jnp.where(cols < (KH + G - 1) // G, v, NEG)
    gis = []
    for _ in range(KN):
        m = jnp.max(v, axis=1, keepdims=True)
        sel = jnp.min(jnp.where(v == m, cols, IMAX), axis=1, keepdims=True)
        gis.append(sel)
        v = jnp.where(cols == sel, NEG, v)
    gi_ref[...] = jnp.concatenate(gis, axis=1)


_gtopk = pl.pallas_call(
    _gtopk_kernel,
    out_shape=jax.ShapeDtypeStruct((B, KN), jnp.int32),
)

GW = (B * KN) // NW
NCH = 10
CHG = GW // NCH


@functools.lru_cache(maxsize=1)
def _sc_group_gather():
    mesh = plsc.VectorSubcoreMesh(core_axis_name="c", subcore_axis_name="s")

    @functools.partial(
        pl.kernel,
        mesh=mesh,
        out_type=jax.ShapeDtypeStruct((B * KN, D * G), jnp.float32),
        scratch_types=[
            pltpu.VMEM((NCH, CHG), jnp.int32),
            pltpu.VMEM((CHG, D * G), jnp.float32),
            pltpu.VMEM((CHG, D * G), jnp.float32),
            pltpu.SemaphoreType.DMA,
        ],
    )
    def gather(table_hbm, idx_hbm, out_hbm, idx_v, b0, b1, sem):
        wid = lax.axis_index("s") * NC + lax.axis_index("c")
        base = wid * GW
        pltpu.sync_copy(idx_hbm.at[wid], idx_v)
        bufs, cps = [b0, b1], [None, None]
        cps[0] = pltpu.async_copy(table_hbm.at[idx_v.at[0]], b0, sem)
        for c in range(1, NCH):
            cps[c % 2] = pltpu.async_copy(table_hbm.at[idx_v.at[c]],
                                          bufs[c % 2], sem)
            cps[(c - 1) % 2].wait()
            pltpu.sync_copy(bufs[(c - 1) % 2],
                            out_hbm.at[pl.ds(base + (c - 1) * CHG, CHG)])
        cps[(NCH - 1) % 2].wait()
        pltpu.sync_copy(bufs[(NCH - 1) % 2],
                        out_hbm.at[pl.ds(base + (NCH - 1) * CHG, CHG)])

    return gather


SB = 640


def _sims_kernel(cand_ref, xl5_ref, dot_ref, hn2_ref):
    c = cand_ref[...].reshape(SB, D, G)
    x3 = xl5_ref[...].reshape(SB, D, 1)
    dot_ref[...] = jnp.sum(c * x3, axis=1)
    hn2_ref[...] = jnp.sum(c * c, axis=1)


_sims = pl.pallas_call(
    _sims_kernel,
    grid=((B * KN) // SB,),
    in_specs=[
        pl.BlockSpec((SB, D * G), lambda i: (i, 0)),
        pl.BlockSpec((SB, D), lambda i: (i, 0)),
    ],
    out_specs=[
        pl.BlockSpec((SB, G), lambda i: (i, 0)),
        pl.BlockSpec((SB, G), lambda i: (i, 0)),
    ],
    out_shape=[
        jax.ShapeDtypeStruct((B * KN, G), jnp.float32),
        jax.ShapeDtypeStruct((B * KN, G), jnp.float32),
    ],
)


def _ftopk_kernel(dot_ref, hn2_ref, gi_ref, xl_ref, tv_ref, ti_ref):
    xl = xl_ref[...]
    xn = jnp.maximum(jnp.sqrt(jnp.sum(xl * xl, axis=1, keepdims=True)), 1e-8)
    hn = jnp.maximum(jnp.sqrt(hn2_ref[...]), 1e-8)
    v = dot_ref[...] / (xn * hn)
    gi = gi_ref[...]
    pos = lax.broadcasted_iota(jnp.int32, (B, KN * G), 1)
    pj = pos // G
    gsel = jnp.zeros((B, KN * G), dtype=jnp.int32)
    for j in range(KN):
        gsel = jnp.where(pj == j, gi[:, j:j + 1], gsel)
    gidx = gsel * G + (pos - pj * G)
    v = jnp.where(gidx < KH, v, NEG)
    tvs, tis = [], []
    for _ in range(KN):
        m = jnp.max(v, axis=1, keepdims=True)
        sel = jnp.min(jnp.where(v == m, gidx, IMAX), axis=1, keepdims=True)
        tvs.append(m)
        tis.append(sel)
        v = jnp.where(gidx == sel, NEG, v)
    tv_ref[...] = jnp.concatenate(tvs, axis=1)
    ti_ref[...] = jnp.concatenate(tis, axis=1)


_ftopk = pl.pallas_call(
    _ftopk_kernel,
    out_shape=[
        jax.ShapeDtypeStruct((B, KN), jnp.float32),
        jax.ShapeDtypeStruct((B, KN), jnp.int32),
    ],
)


GROUP = 128 // D
CH = ROWS_PER_W // 2


@functools.lru_cache(maxsize=1)
def _sc_gather():
    mesh = plsc.VectorSubcoreMesh(core_axis_name="c", subcore_axis_name="s")

    @functools.partial(
        pl.kernel,
        mesh=mesh,
        out_type=jax.ShapeDtypeStruct((B * KN, 128), jnp.float32),
        scratch_types=[
            pltpu.VMEM((2, CH), jnp.int32),
            pltpu.VMEM((ROWS_PER_W, 128), jnp.float32),
            pltpu.SemaphoreType.DMA,
        ],
    )
    def gather(table_hbm, idx_hbm, out_hbm, idx_v, rows_v, sem):
        wid = lax.axis_index("s") * NC + lax.axis_index("c")
        base = wid * ROWS_PER_W
        pltpu.sync_copy(idx_hbm.at[wid], idx_v)
        cp0 = pltpu.async_copy(table_hbm.at[idx_v.at[0]],
                               rows_v.at[pl.ds(0, CH)], sem)
        cp1 = pltpu.async_copy(table_hbm.at[idx_v.at[1]],
                               rows_v.at[pl.ds(CH, CH)], sem)
        cp0.wait()
        cp1.wait()
        pltpu.sync_copy(rows_v, out_hbm.at[pl.ds(base, ROWS_PER_W)])

    return gather


def _fuse_kernel(tv_ref, ti_ref, rows_ref, xl_ref, w1a_ref, w1b_ref, b1_ref,
                 w2_ref, b2_ref, g_ref, bt_ref, o_ref):
    tv = tv_ref[...]
    m = jnp.max(tv, axis=1, keepdims=True)
    e = jnp.exp(tv - m)
    w = e / jnp.sum(e, axis=1, keepdims=True)
    rows = rows_ref[...]
    ti = ti_ref[...]
    ws = jnp.zeros((B, D), dtype=jnp.float32)
    for j in range(KN):
        sub = lax.rem(ti[:, j:j + 1], GROUP)
        hj = jnp.zeros((B, D), dtype=jnp.float32)
        for s in range(GROUP):
            sl = rows[:, j * 128 + s * D:j * 128 + (s + 1) * D]
            hj = jnp.where(sub == s, sl, hj)
        ws = ws + w[:, j:j + 1] * hj
    xl = xl_ref[...]
    dn = (((1,), (0,)), ((), ()))
    h = lax.dot_general(xl, w1a_ref[...], dn,
                        preferred_element_type=jnp.float32,
                        precision=lax.Precision.HIGHEST)
    h = h + lax.dot_general(ws, w1b_ref[...], dn,
                            preferred_element_type=jnp.float32,
                            precision=lax.Precision.HIGHEST)
    h = jnp.maximum(h + b1_ref[...], 0.0)
    h2 = lax.dot_general(h, w2_ref[...], dn,
                         preferred_element_type=jnp.float32,
                         precision=lax.Precision.HIGHEST) + b2_ref[...]
    mu = jnp.mean(h2, axis=1, keepdims=True)
    var = jnp.mean((h2 - mu) ** 2, axis=1, keepdims=True)
    o_ref[...] = (h2 - mu) / jnp.sqrt(var + 1e-5) * g_ref[...] + bt_ref[...]


_fuse = pl.pallas_call(
    _fuse_kernel,
    out_shape=jax.ShapeDtypeStruct((B, D), jnp.float32),
)


def kernel(x, historical_data, W1, b1, W2, b2, gamma, beta):
    xl = x[:, -1, :]
    ht = jnp.pad(historical_data.T, ((0, 0), (0, KHP - KH)))
    gm = _scan(xl, _norm(ht))
    gi = _gtopk(gm.transpose(1, 0, 2).reshape(B, NG))
    table_g = ht.reshape(D, NG, G).transpose(1, 0, 2).reshape(NG, D * G)
    cand = _sc_group_gather()(table_g, gi.reshape(NW, NCH, CHG))
    xl5 = jnp.repeat(xl, KN, axis=0)
    dotc, hn2c = _sims(cand, xl5)
    tv, ti = _ftopk(dotc.reshape(B, KN * G), hn2c.reshape(B, KN * G), gi, xl)
    table8 = historical_data.reshape(KH // GROUP, 128)
    idx_g = (ti // GROUP).reshape(NW, 2, CH)
    rows = _sc_gather()(table8, idx_g)
    w1a = W1[:D]
    w1b = W1[D:].reshape(KN, D, 2 * D).sum(axis=0)
    out = _fuse(tv, ti, rows.reshape(B, KN * 128), xl, w1a, w1b,
                b1.reshape(1, -1), W2, b2.reshape(1, -1),
                gamma.reshape(1, -1), beta.reshape(1, -1))
    return out
